# baseline (speedup 1.0000x reference)
import functools

import jax
import jax.numpy as jnp
from jax import lax
from jax.experimental import pallas as pl
from jax.experimental.pallas import tpu as pltpu
from jax.experimental.pallas import tpu_sc as plsc

N = 262144
D = 12
R = 2048
NB = 128
NF = float(N)
H_PAD = 294912
ZCH = 2048


DP = 16


def _propagate_body(pT_ref, vT_ref, o_ref):
    q = pT_ref[...] + vT_ref[...]
    row = lax.broadcasted_iota(jnp.int32, q.shape, 0)
    is_z = (row % 3) == 2
    q = jnp.where(is_z, q * (q > 0).astype(q.dtype), q)
    qp = jnp.concatenate(
        [q, jnp.zeros((DP - D, q.shape[1]), jnp.float32)], axis=0)
    o_ref[...] = jnp.transpose(qp)


def _propagate(particles, velocity):
    return pl.pallas_call(
        _propagate_body,
        out_shape=jax.ShapeDtypeStruct((N, DP), jnp.float32),
        grid=(32,),
        in_specs=[
            pl.BlockSpec((D, N // 32), lambda i: (0, i)),
            pl.BlockSpec((D, N // 32), lambda i: (0, i)),
        ],
        out_specs=pl.BlockSpec((N // 32, DP), lambda i: (i, 0)),
    )(particles.T, velocity.T)


def _scan_body(s_ref, out_ref, sT_ref, yT_ref, t16_ref, t16T_ref, y2T_ref):
    def acc_body(k, acc):
        return acc + s_ref[pl.ds(8 * k, 8), :]

    acc = lax.fori_loop(1, 256, acc_body, s_ref[0:8, :])
    acc = jnp.roll(acc, -4, axis=0) + acc
    acc = jnp.roll(acc, -2, axis=0) + acc
    acc = jnp.roll(acc, -1, axis=0) + acc
    total = jnp.sum(acc[0:1, :])

    for c in range(16):
        blk = s_ref[pl.ds(128 * c, 128), :] / total
        sT_ref[:, pl.ds(128 * c, 128)] = jnp.transpose(blk)

    row0 = sT_ref[0:1, :]
    yT_ref[0:1, :] = row0

    def scan1(b, carry):
        carry = carry + sT_ref[pl.ds(b, 1), :]
        yT_ref[pl.ds(b, 1), :] = carry
        return carry

    lax.fori_loop(1, 128, scan1, row0)

    t_row = yT_ref[127:128, :]
    for u in range(16):
        t16_ref[pl.ds(u, 1), :] = t_row[:, 128 * u:128 * (u + 1)]
    t16T_ref[...] = jnp.transpose(t16_ref[...])

    r0 = t16T_ref[0:1, :]
    y2T_ref[0:1, :] = r0

    def scan2(v, carry):
        carry = carry + t16T_ref[pl.ds(v, 1), :]
        y2T_ref[pl.ds(v, 1), :] = carry
        return carry

    lax.fori_loop(1, 128, scan2, r0)

    t2 = y2T_ref[127:128, :]
    lane16 = lax.broadcasted_iota(jnp.int32, (1, 16), 1)
    e = jnp.zeros((1, 16), jnp.float32)
    for u in range(1, 16):
        bc = jnp.broadcast_to(t2[:, u - 1:u], (1, 16))
        e = jnp.where(lane16 >= u, e + bc, e)

    PT = y2T_ref[...] + e
    P16 = jnp.transpose(PT)
    p_row = jnp.concatenate([P16[u:u + 1, :] for u in range(16)], axis=1)
    c_row = jnp.concatenate(
        [jnp.zeros((1, 1), jnp.float32), p_row[:, :2047]], axis=1)

    for c in range(16):
        cum = yT_ref[:, pl.ds(128 * c, 128)] + c_row[:, 128 * c:128 * (c + 1)]
        sf = jnp.floor(cum * NF + 0.5)
        sf = jnp.minimum(jnp.maximum(sf, 0.0), NF)
        si = sf.astype(jnp.int32)
        if c == 15:
            rowi = lax.broadcasted_iota(jnp.int32, (128, 128), 0)
            coli = lax.broadcasted_iota(jnp.int32, (128, 128), 1)
            si = jnp.where((rowi == 127) & (coli == 127), N, si)
        out_ref[pl.ds(128 * c, 128), :] = jnp.transpose(si)


def _scan(s2):
    return pl.pallas_call(
        _scan_body,
        out_shape=jax.ShapeDtypeStruct((R, NB), jnp.int32),
        scratch_shapes=[
            pltpu.VMEM((128, R), jnp.float32),
            pltpu.VMEM((128, R), jnp.float32),
            pltpu.VMEM((16, 128), jnp.float32),
            pltpu.VMEM((128, 16), jnp.float32),
            pltpu.VMEM((128, 16), jnp.float32),
        ],
    )(s2)


_SC_MESH = dict(core_axis_name="c", subcore_axis_name="s",
                num_cores=2, num_subcores=16)


def _hist_body(s_hbm, h_hbm, s_vmem, ones_v, zeros_v, h_shared, hsem):
    cid = lax.axis_index("c")
    sid = lax.axis_index("s")
    wid = cid * 16 + sid

    for i in range(8):
        ones_v[pl.ds(16 * i, 16)] = jnp.ones((16,), jnp.int32)
    for i in range(ZCH // 16):
        zeros_v[pl.ds(16 * i, 16)] = jnp.zeros((16,), jnp.int32)

    stripe = H_PAD // 16
    for i in range(stripe // ZCH):
        pltpu.sync_copy(zeros_v,
                        h_shared.at[pl.ds(sid * stripe + i * ZCH, ZCH)])
    plsc.subcore_barrier()

    pltpu.sync_copy(s_hbm.at[pl.ds(wid * 64, 64)], s_vmem)
    adds = [pltpu.async_copy(ones_v, h_shared.at[s_vmem.at[c]], hsem,
                             add=True)
            for c in range(64)]
    for a in adds:
        a.wait()
    plsc.subcore_barrier()

    for i in range(stripe // ZCH):
        off = sid * stripe + i * ZCH
        pltpu.sync_copy(h_shared.at[pl.ds(off, ZCH)],
                        h_hbm.at[pl.ds(cid * H_PAD + off, ZCH)])


_HIST_CACHE = []


def _hist(s_idx):
    if not _HIST_CACHE:
        _HIST_CACHE.append(pl.kernel(
            _hist_body,
            out_type=jax.ShapeDtypeStruct((2 * H_PAD,), jnp.int32),
            mesh=plsc.VectorSubcoreMesh(**_SC_MESH),
            compiler_params=pltpu.CompilerParams(use_tc_tiling_on_sc=False),
            scratch_types=[
                pltpu.VMEM((64, 128), jnp.int32),
                pltpu.VMEM((128,), jnp.int32),
                pltpu.VMEM((ZCH,), jnp.int32),
                pltpu.VMEM_SHARED((H_PAD,), jnp.int32),
                pltpu.SemaphoreType.DMA,
            ],
        ))
    return _HIST_CACHE[0](s_idx)


def _hs_prefix_128(x):
    lane = lax.broadcasted_iota(jnp.int32, x.shape, 1)
    for d in (1, 2, 4, 8, 16, 32, 64):
        x = x + jnp.where(lane >= d, jnp.roll(x, d, axis=1), 0.0)
    return x


def _prefix_body(h_ref, out_ref, ps_ref, rs_ref):
    for rblk in range(16):
        h = (h_ref[0, pl.ds(128 * rblk, 128), :]
             + h_ref[1, pl.ds(128 * rblk, 128), :]).astype(jnp.float32)
        y = _hs_prefix_128(h)
        ps_ref[pl.ds(128 * rblk, 128), :] = y
        rs_ref[:, pl.ds(128 * rblk, 128)] = jnp.transpose(y[:, 127:128])

    g16 = jnp.concatenate(
        [rs_ref[:, pl.ds(128 * u, 128)] for u in range(16)], axis=0)
    incl16 = _hs_prefix_128(g16)
    subl = lax.broadcasted_iota(jnp.int32, (16, 1), 0)
    e16 = jnp.zeros((16, 1), jnp.float32)
    for u in range(1, 16):
        bc = jnp.broadcast_to(incl16[u - 1:u, 127:128], (16, 1))
        e16 = jnp.where(subl >= u, e16 + bc, e16)
    p16 = incl16 + e16
    p_row = jnp.concatenate([p16[u:u + 1, :] for u in range(16)], axis=1)
    excl = jnp.concatenate(
        [jnp.zeros((1, 1), jnp.float32), p_row[:, :2047]], axis=1)

    for rblk in range(16):
        carry = jnp.transpose(excl[:, 128 * rblk:128 * (rblk + 1)])
        idxf = ps_ref[pl.ds(128 * rblk, 128), :] + carry
        idxf = jnp.minimum(idxf, float(N - 1))
        out_ref[pl.ds(128 * rblk, 128), :] = idxf.astype(jnp.int32)


def _prefix(h3):
    return pl.pallas_call(
        _prefix_body,
        out_shape=jax.ShapeDtypeStruct((R, NB), jnp.int32),
        scratch_shapes=[
            pltpu.VMEM((R, NB), jnp.float32),
            pltpu.VMEM((1, R), jnp.float32),
        ],
    )(h3)


def _gather_body(q_hbm, idx_hbm, out_hbm, idx_v, rows_v, sem0, sem1):
    cid = lax.axis_index("c")
    sid = lax.axis_index("s")
    wid = cid * 16 + sid

    pltpu.sync_copy(idx_hbm.at[pl.ds(wid * 64, 64)], idx_v)
    bufs = (rows_v.at[0], rows_v.at[1])
    sems = (sem0, sem1)
    pltpu.async_copy(q_hbm.at[idx_v.at[0]], bufs[0], sems[0])
    pltpu.async_copy(q_hbm.at[idx_v.at[1]], bufs[1], sems[1])
    for c in range(64):
        b = c & 1
        pltpu.make_async_copy(q_hbm.at[idx_v.at[c]], bufs[b], sems[b]).wait()
        pltpu.sync_copy(bufs[b],
                        out_hbm.at[pl.ds(wid * 8192 + c * 128, 128)])
        if c + 2 < 64:
            pltpu.async_copy(q_hbm.at[idx_v.at[c + 2]], bufs[b], sems[b])


_GATHER_CACHE = []


def _gather(q, idx):
    if not _GATHER_CACHE:
        _GATHER_CACHE.append(pl.kernel(
            _gather_body,
            out_type=jax.ShapeDtypeStruct((N, DP), jnp.float32),
            mesh=plsc.VectorSubcoreMesh(**_SC_MESH),
            compiler_params=pltpu.CompilerParams(use_tc_tiling_on_sc=False),
            scratch_types=[
                pltpu.VMEM((64, 128), jnp.int32),
                pltpu.VMEM((2, 128, DP), jnp.float32),
                pltpu.SemaphoreType.DMA,
                pltpu.SemaphoreType.DMA,
            ],
        ))
    return _GATHER_CACHE[0](q, idx)


def kernel(particles, velocity, scores):
    s2 = scores.reshape(R, NB)
    s_idx = _scan(s2)
    h = _hist(s_idx).reshape(2, H_PAD)
    q = _propagate(particles, velocity)
    h3 = h[:, :N].reshape(2, R, NB)
    idx = _prefix(h3)
    out16 = _gather(q, idx)
    return out16[:, :D]


# speedup vs baseline: 1.0394x; 1.0394x over previous
import functools

import jax
import jax.numpy as jnp
from jax import lax
from jax.experimental import pallas as pl
from jax.experimental.pallas import tpu as pltpu
from jax.experimental.pallas import tpu_sc as plsc

N = 262144
D = 12
R = 2048
NB = 128
NF = float(N)
H_PAD = 294912
ZCH = 2048


DP = 16


def _propagate_body(pT_ref, vT_ref, o_ref):
    q = pT_ref[...] + vT_ref[...]
    row = lax.broadcasted_iota(jnp.int32, q.shape, 0)
    is_z = (row % 3) == 2
    q = jnp.where(is_z, q * (q > 0).astype(q.dtype), q)
    qp = jnp.concatenate(
        [q, jnp.zeros((DP - D, q.shape[1]), jnp.float32)], axis=0)
    o_ref[...] = jnp.transpose(qp)


def _propagate(particles, velocity):
    return pl.pallas_call(
        _propagate_body,
        out_shape=jax.ShapeDtypeStruct((N, DP), jnp.float32),
        grid=(32,),
        in_specs=[
            pl.BlockSpec((D, N // 32), lambda i: (0, i)),
            pl.BlockSpec((D, N // 32), lambda i: (0, i)),
        ],
        out_specs=pl.BlockSpec((N // 32, DP), lambda i: (i, 0)),
    )(particles.T, velocity.T)


def _scan_body(s_ref, out_ref, sT_ref, yT_ref, t16_ref, t16T_ref, y2T_ref):
    def acc_body(k, acc):
        return acc + s_ref[pl.ds(8 * k, 8), :]

    acc = lax.fori_loop(1, 256, acc_body, s_ref[0:8, :])
    acc = jnp.roll(acc, -4, axis=0) + acc
    acc = jnp.roll(acc, -2, axis=0) + acc
    acc = jnp.roll(acc, -1, axis=0) + acc
    total = jnp.sum(acc[0:1, :])

    for c in range(16):
        blk = s_ref[pl.ds(128 * c, 128), :] / total
        sT_ref[:, pl.ds(128 * c, 128)] = jnp.transpose(blk)

    row0 = sT_ref[0:1, :]
    yT_ref[0:1, :] = row0

    def scan1(b, carry):
        carry = carry + sT_ref[pl.ds(b, 1), :]
        yT_ref[pl.ds(b, 1), :] = carry
        return carry

    lax.fori_loop(1, 128, scan1, row0)

    t_row = yT_ref[127:128, :]
    for u in range(16):
        t16_ref[pl.ds(u, 1), :] = t_row[:, 128 * u:128 * (u + 1)]
    t16T_ref[...] = jnp.transpose(t16_ref[...])

    r0 = t16T_ref[0:1, :]
    y2T_ref[0:1, :] = r0

    def scan2(v, carry):
        carry = carry + t16T_ref[pl.ds(v, 1), :]
        y2T_ref[pl.ds(v, 1), :] = carry
        return carry

    lax.fori_loop(1, 128, scan2, r0)

    t2 = y2T_ref[127:128, :]
    lane16 = lax.broadcasted_iota(jnp.int32, (1, 16), 1)
    e = jnp.zeros((1, 16), jnp.float32)
    for u in range(1, 16):
        bc = jnp.broadcast_to(t2[:, u - 1:u], (1, 16))
        e = jnp.where(lane16 >= u, e + bc, e)

    PT = y2T_ref[...] + e
    P16 = jnp.transpose(PT)
    p_row = jnp.concatenate([P16[u:u + 1, :] for u in range(16)], axis=1)
    c_row = jnp.concatenate(
        [jnp.zeros((1, 1), jnp.float32), p_row[:, :2047]], axis=1)

    for c in range(16):
        cum = yT_ref[:, pl.ds(128 * c, 128)] + c_row[:, 128 * c:128 * (c + 1)]
        sf = jnp.floor(cum * NF + 0.5)
        sf = jnp.minimum(jnp.maximum(sf, 0.0), NF)
        si = sf.astype(jnp.int32)
        if c == 15:
            rowi = lax.broadcasted_iota(jnp.int32, (128, 128), 0)
            coli = lax.broadcasted_iota(jnp.int32, (128, 128), 1)
            si = jnp.where((rowi == 127) & (coli == 127), N, si)
        out_ref[pl.ds(128 * c, 128), :] = jnp.transpose(si)


def _scan(s2):
    return pl.pallas_call(
        _scan_body,
        out_shape=jax.ShapeDtypeStruct((R, NB), jnp.int32),
        scratch_shapes=[
            pltpu.VMEM((128, R), jnp.float32),
            pltpu.VMEM((128, R), jnp.float32),
            pltpu.VMEM((16, 128), jnp.float32),
            pltpu.VMEM((128, 16), jnp.float32),
            pltpu.VMEM((128, 16), jnp.float32),
        ],
    )(s2)


_SC_MESH = dict(core_axis_name="c", subcore_axis_name="s",
                num_cores=2, num_subcores=16)


def _hist_body(s_hbm, h_hbm, s_vmem, ones_v, zeros_v, h_shared, hsem):
    cid = lax.axis_index("c")
    sid = lax.axis_index("s")
    wid = cid * 16 + sid

    for i in range(8):
        ones_v[pl.ds(16 * i, 16)] = jnp.ones((16,), jnp.int32)
    for i in range(ZCH // 16):
        zeros_v[pl.ds(16 * i, 16)] = jnp.zeros((16,), jnp.int32)

    stripe = H_PAD // 16
    for i in range(stripe // ZCH):
        pltpu.sync_copy(zeros_v,
                        h_shared.at[pl.ds(sid * stripe + i * ZCH, ZCH)])
    plsc.subcore_barrier()

    pltpu.sync_copy(s_hbm.at[pl.ds(wid * 64, 64)], s_vmem)
    adds = [pltpu.async_copy(ones_v, h_shared.at[s_vmem.at[c]], hsem,
                             add=True)
            for c in range(64)]
    for a in adds:
        a.wait()
    plsc.subcore_barrier()

    for i in range(stripe // ZCH):
        off = sid * stripe + i * ZCH
        pltpu.sync_copy(h_shared.at[pl.ds(off, ZCH)],
                        h_hbm.at[pl.ds(cid * H_PAD + off, ZCH)])


_HIST_CACHE = []


def _hist(s_idx):
    if not _HIST_CACHE:
        _HIST_CACHE.append(pl.kernel(
            _hist_body,
            out_type=jax.ShapeDtypeStruct((2 * H_PAD,), jnp.int32),
            mesh=plsc.VectorSubcoreMesh(**_SC_MESH),
            compiler_params=pltpu.CompilerParams(use_tc_tiling_on_sc=False),
            scratch_types=[
                pltpu.VMEM((64, 128), jnp.int32),
                pltpu.VMEM((128,), jnp.int32),
                pltpu.VMEM((ZCH,), jnp.int32),
                pltpu.VMEM_SHARED((H_PAD,), jnp.int32),
                pltpu.SemaphoreType.DMA,
            ],
        ))
    return _HIST_CACHE[0](s_idx)


def _hs_prefix_128(x):
    lane = lax.broadcasted_iota(jnp.int32, x.shape, 1)
    for d in (1, 2, 4, 8, 16, 32, 64):
        x = x + jnp.where(lane >= d, jnp.roll(x, d, axis=1), 0.0)
    return x


def _prefix_body(h_ref, out_ref, ps_ref, rs_ref):
    for rblk in range(16):
        h = (h_ref[0, pl.ds(128 * rblk, 128), :]
             + h_ref[1, pl.ds(128 * rblk, 128), :]).astype(jnp.float32)
        y = _hs_prefix_128(h)
        ps_ref[pl.ds(128 * rblk, 128), :] = y
        rs_ref[:, pl.ds(128 * rblk, 128)] = jnp.transpose(y[:, 127:128])

    g16 = jnp.concatenate(
        [rs_ref[:, pl.ds(128 * u, 128)] for u in range(16)], axis=0)
    incl16 = _hs_prefix_128(g16)
    subl = lax.broadcasted_iota(jnp.int32, (16, 1), 0)
    e16 = jnp.zeros((16, 1), jnp.float32)
    for u in range(1, 16):
        bc = jnp.broadcast_to(incl16[u - 1:u, 127:128], (16, 1))
        e16 = jnp.where(subl >= u, e16 + bc, e16)
    p16 = incl16 + e16
    p_row = jnp.concatenate([p16[u:u + 1, :] for u in range(16)], axis=1)
    excl = jnp.concatenate(
        [jnp.zeros((1, 1), jnp.float32), p_row[:, :2047]], axis=1)

    for rblk in range(16):
        carry = jnp.transpose(excl[:, 128 * rblk:128 * (rblk + 1)])
        idxf = ps_ref[pl.ds(128 * rblk, 128), :] + carry
        idxf = jnp.minimum(idxf, float(N - 1))
        out_ref[pl.ds(128 * rblk, 128), :] = idxf.astype(jnp.int32)


def _prefix(h3):
    return pl.pallas_call(
        _prefix_body,
        out_shape=jax.ShapeDtypeStruct((R, NB), jnp.int32),
        scratch_shapes=[
            pltpu.VMEM((R, NB), jnp.float32),
            pltpu.VMEM((1, R), jnp.float32),
        ],
    )(h3)


def _gather_body(q_hbm, idx_hbm, out_hbm, idx_v, rows_v,
                 sem0, sem1, sem2, sem3):
    cid = lax.axis_index("c")
    sid = lax.axis_index("s")
    wid = cid * 16 + sid

    pltpu.sync_copy(idx_hbm.at[pl.ds(wid * 64, 64)], idx_v)
    bufs = (rows_v.at[0], rows_v.at[1], rows_v.at[2], rows_v.at[3])
    sems = (sem0, sem1, sem2, sem3)
    for k in range(4):
        pltpu.async_copy(q_hbm.at[idx_v.at[k]], bufs[k], sems[k])
    for c in range(64):
        b = c & 3
        pltpu.make_async_copy(q_hbm.at[idx_v.at[c]], bufs[b], sems[b]).wait()
        pltpu.sync_copy(bufs[b],
                        out_hbm.at[pl.ds(wid * 8192 + c * 128, 128)])
        if c + 4 < 64:
            pltpu.async_copy(q_hbm.at[idx_v.at[c + 4]], bufs[b], sems[b])


_GATHER_CACHE = []


def _gather(q, idx):
    if not _GATHER_CACHE:
        _GATHER_CACHE.append(pl.kernel(
            _gather_body,
            out_type=jax.ShapeDtypeStruct((N, DP), jnp.float32),
            mesh=plsc.VectorSubcoreMesh(**_SC_MESH),
            compiler_params=pltpu.CompilerParams(use_tc_tiling_on_sc=False),
            scratch_types=[
                pltpu.VMEM((64, 128), jnp.int32),
                pltpu.VMEM((4, 128, DP), jnp.float32),
                pltpu.SemaphoreType.DMA,
                pltpu.SemaphoreType.DMA,
                pltpu.SemaphoreType.DMA,
                pltpu.SemaphoreType.DMA,
            ],
        ))
    return _GATHER_CACHE[0](q, idx)


def kernel(particles, velocity, scores):
    s2 = scores.reshape(R, NB)
    s_idx = _scan(s2)
    h = _hist(s_idx).reshape(2, H_PAD)
    q = _propagate(particles, velocity)
    h3 = h[:, :N].reshape(2, R, NB)
    idx = _prefix(h3)
    out16 = _gather(q, idx)
    return out16[:, :D]
